# baseline (speedup 1.0000x reference)
import functools

import jax
import jax.numpy as jnp
from jax import lax
from jax.experimental import pallas as pl
from jax.experimental.pallas import tpu as pltpu
from jax.experimental.pallas import tpu_sc as plsc

_N_FINE = 200000
_N_COARSE = 50000
_K = 9
_F = 128
_N_IDX = _N_COARSE * _K

_info = plsc.get_sparse_core_info()
_NC = _info.num_cores
_NS = _info.num_subcores
_NW = _NC * _NS

_P = 5
_SLICE_C = _N_COARSE // _P
_SLICE_IDX = _SLICE_C * _K

_CH = 240
_N_CHUNKS = _SLICE_IDX // _CH

_mesh = plsc.VectorSubcoreMesh(core_axis_name="c", subcore_axis_name="s")


@functools.partial(
    pl.kernel,
    mesh=_mesh,
    out_type=jax.ShapeDtypeStruct((_SLICE_IDX, _F), jnp.float32),
    scratch_types=[
        pltpu.VMEM((_CH,), jnp.int32),
        pltpu.VMEM((_CH, _F), jnp.float32),
        pltpu.SemaphoreType.DMA,
    ],
)
def _sc_gather(lv_hbm, idx_hbm, out_hbm, idx_v, rows_v, sem):
    wid = lax.axis_index("s") * _NC + lax.axis_index("c")
    n_mine = (_N_CHUNKS - wid + _NW - 1) // _NW

    def body(i, carry):
        base = (wid + i * _NW) * _CH
        pltpu.sync_copy(idx_hbm.at[pl.ds(base, _CH)], idx_v)
        pltpu.async_copy(lv_hbm.at[idx_v], rows_v, sem).wait()
        pltpu.sync_copy(rows_v, out_hbm.at[pl.ds(base, _CH)])
        return carry

    lax.fori_loop(0, n_mine, body, 0)


_BM = 2000


def _mm_body(g_ref, w_ref, b_ref, o_ref):
    k = pl.program_id(1)

    @pl.when(k == 0)
    def _init():
        o_ref[...] = jnp.broadcast_to(b_ref[...], (_BM, _F))

    o_ref[...] += jnp.dot(g_ref[0], w_ref[0],
                          preferred_element_type=jnp.float32)

    @pl.when(k == _K - 1)
    def _relu():
        o_ref[...] = jnp.maximum(o_ref[...], 0.0)


def _tc_matmul(g3, w3, b2d):
    return pl.pallas_call(
        _mm_body,
        grid=(_SLICE_C // _BM, _K),
        in_specs=[
            pl.BlockSpec((1, _BM, _F), lambda i, k: (k, i, 0)),
            pl.BlockSpec((1, _F, _F), lambda i, k: (k, 0, 0)),
            pl.BlockSpec((1, _F), lambda i, k: (0, 0)),
        ],
        out_specs=pl.BlockSpec((_BM, _F), lambda i, k: (i, 0)),
        out_shape=jax.ShapeDtypeStruct((_SLICE_C, _F), jnp.float32),
    )(g3, w3, b2d)


def kernel(lv, ls_neighbors, W, b):
    idx_t = ls_neighbors.astype(jnp.int32).T
    w3 = W.reshape(_K, _F, _F)
    b2d = b.reshape(1, _F)
    outs = []
    for p in range(_P):
        idx_p = lax.slice_in_dim(idx_t, p * _SLICE_C, (p + 1) * _SLICE_C,
                                 axis=1).reshape(_SLICE_IDX)
        g = _sc_gather(lv, idx_p)
        g3 = g.reshape(_K, _SLICE_C, _F)
        outs.append(_tc_matmul(g3, w3, b2d))
    out = jnp.concatenate(outs, axis=0)
    return (out, ls_neighbors)


# speedup vs baseline: 1.0161x; 1.0161x over previous
import functools

import jax
import jax.numpy as jnp
from jax import lax
from jax.experimental import pallas as pl
from jax.experimental.pallas import tpu as pltpu
from jax.experimental.pallas import tpu_sc as plsc

_N_FINE = 200000
_N_COARSE = 50000
_K = 9
_F = 128
_N_IDX = _N_COARSE * _K

_info = plsc.get_sparse_core_info()
_NC = _info.num_cores
_NS = _info.num_subcores
_NW = _NC * _NS

_P = 5
_SLICE_C = _N_COARSE // _P
_SLICE_IDX = _SLICE_C * _K

_CH = 240
_N_CHUNKS = _SLICE_IDX // _CH
_MAXC = -(-_N_CHUNKS // _NW)

_mesh = plsc.VectorSubcoreMesh(core_axis_name="c", subcore_axis_name="s")


@functools.partial(
    pl.kernel,
    mesh=_mesh,
    out_type=jax.ShapeDtypeStruct((_SLICE_IDX, _F), jnp.float32),
    scratch_types=[
        pltpu.VMEM((_MAXC * _CH,), jnp.int32),
        pltpu.VMEM((_CH, _F), jnp.float32),
        pltpu.VMEM((_CH, _F), jnp.float32),
        pltpu.SemaphoreType.DMA,
        pltpu.SemaphoreType.DMA,
        pltpu.SemaphoreType.DMA,
        pltpu.SemaphoreType.DMA,
    ],
)
def _sc_gather(lv_hbm, idx_hbm, out_hbm, idx_v, buf0, buf1,
               gsem0, gsem1, wsem0, wsem1):
    wid = lax.axis_index("s") * _NC + lax.axis_index("c")
    c0 = wid * _N_CHUNKS // _NW
    n = (wid + 1) * _N_CHUNKS // _NW - c0
    base0 = c0 * _CH
    pltpu.sync_copy(idx_hbm.at[pl.ds(base0, _MAXC * _CH)], idx_v)

    def _start_gather(i, buf, gsem):
        pltpu.async_copy(
            lv_hbm.at[idx_v.at[pl.ds(i * _CH, _CH)]], buf, gsem)

    def _wait_gather(buf, gsem):
        pltpu.make_async_copy(out_hbm.at[pl.ds(0, _CH)], buf, gsem).wait()

    def _start_write(i, buf, wsem):
        pltpu.async_copy(buf, out_hbm.at[pl.ds(base0 + i * _CH, _CH)], wsem)

    def _wait_write(buf, wsem):
        pltpu.make_async_copy(buf, out_hbm.at[pl.ds(0, _CH)], wsem).wait()

    _start_gather(0, buf0, gsem0)

    def body(i, carry):
        even = lax.rem(i, 2) == 0

        @pl.when(even)
        def _even():
            @pl.when(i + 1 < n)
            def _next():
                @pl.when(i >= 1)
                def _w():
                    _wait_write(buf1, wsem1)
                _start_gather(i + 1, buf1, gsem1)
            _wait_gather(buf0, gsem0)
            _start_write(i, buf0, wsem0)

        @pl.when(jnp.logical_not(even))
        def _odd():
            @pl.when(i + 1 < n)
            def _next():
                _wait_write(buf0, wsem0)
                _start_gather(i + 1, buf0, gsem0)
            _wait_gather(buf1, gsem1)
            _start_write(i, buf1, wsem1)

        return carry

    lax.fori_loop(0, n, body, 0)
    _wait_write(buf0, wsem0)
    _wait_write(buf1, wsem1)


_BM = 2000


def _mm_body(g_ref, w_ref, b_ref, o_ref):
    k = pl.program_id(1)

    @pl.when(k == 0)
    def _init():
        o_ref[...] = jnp.broadcast_to(b_ref[...], (_BM, _F))

    o_ref[...] += jnp.dot(g_ref[0], w_ref[0],
                          preferred_element_type=jnp.float32)

    @pl.when(k == _K - 1)
    def _relu():
        o_ref[...] = jnp.maximum(o_ref[...], 0.0)


def _tc_matmul(g3, w3, b2d):
    return pl.pallas_call(
        _mm_body,
        grid=(_SLICE_C // _BM, _K),
        in_specs=[
            pl.BlockSpec((1, _BM, _F), lambda i, k: (k, i, 0)),
            pl.BlockSpec((1, _F, _F), lambda i, k: (k, 0, 0)),
            pl.BlockSpec((1, _F), lambda i, k: (0, 0)),
        ],
        out_specs=pl.BlockSpec((_BM, _F), lambda i, k: (i, 0)),
        out_shape=jax.ShapeDtypeStruct((_SLICE_C, _F), jnp.float32),
    )(g3, w3, b2d)


def kernel(lv, ls_neighbors, W, b):
    idx_t = ls_neighbors.astype(jnp.int32).T
    w3 = W.reshape(_K, _F, _F)
    b2d = b.reshape(1, _F)
    outs = []
    for p in range(_P):
        idx_p = lax.slice_in_dim(idx_t, p * _SLICE_C, (p + 1) * _SLICE_C,
                                 axis=1).reshape(_SLICE_IDX)
        g = _sc_gather(lv, idx_p)
        g3 = g.reshape(_K, _SLICE_C, _F)
        outs.append(_tc_matmul(g3, w3, b2d))
    out = jnp.concatenate(outs, axis=0)
    return (out, ls_neighbors)


# speedup vs baseline: 1.0170x; 1.0009x over previous
import functools

import jax
import jax.numpy as jnp
from jax import lax
from jax.experimental import pallas as pl
from jax.experimental.pallas import tpu as pltpu
from jax.experimental.pallas import tpu_sc as plsc

_N_FINE = 200000
_N_COARSE = 50000
_K = 9
_F = 128
_N_IDX = _N_COARSE * _K

_info = plsc.get_sparse_core_info()
_NC = _info.num_cores
_NS = _info.num_subcores
_NW = _NC * _NS

_P = 5
_SLICE_C = _N_COARSE // _P
_SLICE_IDX = _SLICE_C * _K

_CH = 240
_N_CHUNKS = _SLICE_IDX // _CH
_MAXC = -(-_N_CHUNKS // _NW)

_mesh = plsc.VectorSubcoreMesh(core_axis_name="c", subcore_axis_name="s")


@functools.partial(
    pl.kernel,
    mesh=_mesh,
    out_type=jax.ShapeDtypeStruct((_SLICE_IDX, _F), jnp.float32),
    scratch_types=[
        pltpu.VMEM((_MAXC * _CH,), jnp.int32),
        pltpu.VMEM((_CH, _F), jnp.float32),
        pltpu.VMEM((_CH, _F), jnp.float32),
        pltpu.SemaphoreType.DMA,
        pltpu.SemaphoreType.DMA,
        pltpu.SemaphoreType.DMA,
        pltpu.SemaphoreType.DMA,
    ],
)
def _sc_gather(lv_hbm, idx_hbm, out_hbm, idx_v, buf0, buf1,
               gsem0, gsem1, wsem0, wsem1):
    wid = lax.axis_index("s") * _NC + lax.axis_index("c")
    c0 = wid * _N_CHUNKS // _NW
    n = (wid + 1) * _N_CHUNKS // _NW - c0
    base0 = c0 * _CH
    pltpu.sync_copy(idx_hbm.at[pl.ds(base0, _MAXC * _CH)], idx_v)

    def _start_gather(i, buf, gsem):
        pltpu.async_copy(
            lv_hbm.at[idx_v.at[pl.ds(i * _CH, _CH)]], buf, gsem)

    def _wait_gather(buf, gsem):
        pltpu.make_async_copy(out_hbm.at[pl.ds(0, _CH)], buf, gsem).wait()

    def _start_write(i, buf, wsem):
        pltpu.async_copy(buf, out_hbm.at[pl.ds(base0 + i * _CH, _CH)], wsem)

    def _wait_write(buf, wsem):
        pltpu.make_async_copy(buf, out_hbm.at[pl.ds(0, _CH)], wsem).wait()

    _start_gather(0, buf0, gsem0)

    def body(i, carry):
        even = lax.rem(i, 2) == 0

        @pl.when(even)
        def _even():
            @pl.when(i + 1 < n)
            def _next():
                @pl.when(i >= 1)
                def _w():
                    _wait_write(buf1, wsem1)
                _start_gather(i + 1, buf1, gsem1)
            _wait_gather(buf0, gsem0)
            _start_write(i, buf0, wsem0)

        @pl.when(jnp.logical_not(even))
        def _odd():
            @pl.when(i + 1 < n)
            def _next():
                _wait_write(buf0, wsem0)
                _start_gather(i + 1, buf0, gsem0)
            _wait_gather(buf1, gsem1)
            _start_write(i, buf1, wsem1)

        return carry

    lax.fori_loop(0, n, body, 0)
    _wait_write(buf0, wsem0)
    _wait_write(buf1, wsem1)


_BM = 2000


def _mm_body(g_ref, w_ref, b_ref, o_ref):
    k = pl.program_id(1)

    @pl.when(k == 0)
    def _init():
        o_ref[...] = jnp.broadcast_to(b_ref[...], (_BM, _F))

    o_ref[...] += jnp.dot(g_ref[0], w_ref[0],
                          preferred_element_type=jnp.float32)

    @pl.when(k == _K - 1)
    def _relu():
        o_ref[...] = jnp.maximum(o_ref[...], 0.0)


def _tc_matmul(g3, w3, b2d):
    return pl.pallas_call(
        _mm_body,
        grid=(_SLICE_C // _BM, _K),
        in_specs=[
            pl.BlockSpec((1, _BM, _F), lambda i, k: (k, i, 0)),
            pl.BlockSpec((1, _F, _F), lambda i, k: (k, 0, 0)),
            pl.BlockSpec((1, _F), lambda i, k: (0, 0)),
        ],
        out_specs=pl.BlockSpec((_BM, _F), lambda i, k: (i, 0)),
        out_shape=jax.ShapeDtypeStruct((_SLICE_C, _F), jnp.float32),
    )(g3, w3, b2d)


def kernel(lv, ls_neighbors, W, b):
    idx_t = ls_neighbors.astype(jnp.int32).T
    w3 = W.reshape(_K, _F, _F)
    b2d = b.reshape(1, _F)
    gs = []
    for p in range(_P):
        idx_p = lax.slice_in_dim(idx_t, p * _SLICE_C, (p + 1) * _SLICE_C,
                                 axis=1).reshape(_SLICE_IDX)
        gs.append(_sc_gather(lv, idx_p).reshape(_K, _SLICE_C, _F))
    outs = [_tc_matmul(g3, w3, b2d) for g3 in gs]
    out = jnp.concatenate(outs, axis=0)
    return (out, ls_neighbors)
